# baseline (speedup 1.0000x reference)
import functools

import jax
import jax.numpy as jnp
from jax import lax
from jax.experimental import pallas as pl
from jax.experimental.pallas import tpu as pltpu
from jax.experimental.pallas import tpu_sc as plsc

B = 1024
L = 200
D = 128
LANES = 16
ND = D // LANES
NCHUNK = 13
LP = NCHUNK * LANES
NC = 2
NS = 16
NW = NC * NS
ROWS_PER = B // NW
IDX_SPLIT = 2
IDX_MINOR = L // IDX_SPLIT

_LANE = None


def _mean_pass(g1):
    SUM_UNROLL = 4
    zeros8 = tuple(jnp.zeros((LANES,), jnp.float32) for _ in range(ND))

    @plsc.parallel_loop(0, L, SUM_UNROLL, unroll=2, carry=zeros8)
    def sum_body(l0, acc):
        for t in range(SUM_UNROLL):
            l = l0 + t
            acc = tuple(acc[j] + g1[l, pl.ds(LANES * j, LANES)]
                        for j in range(ND))
        return acc

    return tuple(x * jnp.float32(1.0 / L) for x in sum_body)


def _xlane_reduce(pm, lane, v, op):
    for s in (8, 4, 2, 1):
        plsc.store_scatter(pm, [lane], v)
        shuf = plsc.load_gather(pm, [lane ^ s])
        v = op(v, shuf)
    return v


def _score_pass(ga, u, sc_v, pm, lane):
    mneg = jnp.full((LANES,), -1e30, jnp.float32)

    @plsc.parallel_loop(0, NCHUNK, 1, carry=mneg)
    def score_chunk(c, mv):
        poff = c * (17 * LANES)
        for k in range(LANES):
            l = c * LANES + k
            p = ga[l, pl.ds(0, LANES)] * u[0]
            for j in range(1, ND):
                p = p + ga[l, pl.ds(LANES * j, LANES)] * u[j]
            plsc.store_scatter(pm, [lane + (17 * k) + poff], p)
        sv = plsc.load_gather(pm, [lane * 17 + poff])
        for cc in range(1, LANES):
            sv = sv + plsc.load_gather(pm, [lane * 17 + cc + poff])
        valid = (c * LANES + lane) < L
        sv = jnp.where(valid, sv, jnp.float32(-1e30))
        sc_v[pl.ds(c * LANES, LANES)] = sv
        return jnp.maximum(mv, sv)

    return score_chunk


def _exp_pass(sc_v, pm, lane, mv):
    m = _xlane_reduce(pm, lane, mv, jnp.maximum)

    @plsc.parallel_loop(0, NCHUNK, 1, unroll=2,
                        carry=jnp.zeros((LANES,), jnp.float32))
    def exp_chunk(c, zv):
        e = jnp.exp(sc_v[pl.ds(c * LANES, LANES)] - m)
        sc_v[pl.ds(c * LANES, LANES)] = e
        return zv + e

    return jnp.float32(1.0) / _xlane_reduce(pm, lane, exp_chunk, jnp.add)


def _weighted_pass(gc, sc_v, zinv, u):
    zeros8 = tuple(jnp.zeros((LANES,), jnp.float32) for _ in range(ND))

    @plsc.parallel_loop(0, NCHUNK, 1, carry=zeros8)
    def w_chunk(c, acc):
        e = sc_v[pl.ds(c * LANES, LANES)]
        for k in range(LANES):
            l = c * LANES + k
            w = e[k]
            acc = tuple(acc[j] + w * gc[l, pl.ds(LANES * j, LANES)]
                        for j in range(ND))
        return acc

    return tuple(u[j] + w_chunk[j] * zinv for j in range(ND))


@functools.partial(
    pl.kernel,
    mesh=plsc.VectorSubcoreMesh(core_axis_name="c", subcore_axis_name="s"),
    compiler_params=pltpu.CompilerParams(needs_layout_passes=False),
    out_type=jax.ShapeDtypeStruct((B, D), jnp.float32),
    scratch_types=[
        pltpu.VMEM((2, IDX_SPLIT, IDX_MINOR), jnp.int32),
        pltpu.VMEM((LP, D), jnp.float32),
        pltpu.VMEM((LP, D), jnp.float32),
        pltpu.VMEM((LP, D), jnp.float32),
        pltpu.VMEM((LP,), jnp.float32),
        pltpu.VMEM((NCHUNK * LANES * 17,), jnp.float32),
        pltpu.VMEM((ROWS_PER, D), jnp.float32),
        pltpu.SemaphoreType.DMA,
        pltpu.SemaphoreType.DMA,
        pltpu.SemaphoreType.DMA,
        pltpu.SemaphoreType.DMA,
    ],
)
def _encoder_sc(story_hbm, c1, c2, c3, out_hbm,
                idx2, g1, g2, g3, sc_v, pm, uout_v,
                sem_g1, sem_g2, sem_g3, sem_idx):
    wid = lax.axis_index("s") * NC + lax.axis_index("c")
    base = wid * ROWS_PER
    lane = lax.iota(jnp.int32, LANES)

    zvec = jnp.zeros((LANES,), jnp.float32)
    for g in (g1, g2, g3):
        for r in range(L, LP):
            for j in range(ND):
                g[r, pl.ds(LANES * j, LANES)] = zvec

    def g_issue(tab, g, sem, p):
        for h in range(IDX_SPLIT):
            pltpu.async_copy(
                tab.at[idx2.at[p, h]],
                g.at[pl.ds(h * IDX_MINOR, IDX_MINOR)], sem)

    def g_wait(tab, g, sem, p):
        for h in range(IDX_SPLIT):
            pltpu.make_async_copy(
                tab.at[idx2.at[p, h]],
                g.at[pl.ds(h * IDX_MINOR, IDX_MINOR)], sem).wait()

    def idx_wait(b, p):
        pltpu.make_async_copy(story_hbm.at[b], idx2.at[p], sem_idx).wait()

    pltpu.sync_copy(story_hbm.at[base], idx2.at[0])
    g_issue(c1, g1, sem_g1, 0)
    g_issue(c2, g2, sem_g2, 0)
    g_issue(c3, g3, sem_g3, 0)
    pltpu.async_copy(story_hbm.at[base + 1], idx2.at[1], sem_idx)

    def row_body(r, carry):
        p = r & 1
        pn = 1 - p
        b1 = base + jnp.minimum(r + 1, ROWS_PER - 1)
        b2 = base + jnp.minimum(r + 2, ROWS_PER - 1)

        g_wait(c1, g1, sem_g1, p)
        u = _mean_pass(g1)
        mv = _score_pass(g1, u, sc_v, pm, lane)
        idx_wait(b1, pn)
        g_issue(c1, g1, sem_g1, pn)
        zinv = _exp_pass(sc_v, pm, lane, mv)

        g_wait(c2, g2, sem_g2, p)
        u = _weighted_pass(g2, sc_v, zinv, u)
        mv = _score_pass(g2, u, sc_v, pm, lane)
        g_issue(c2, g2, sem_g2, pn)
        zinv = _exp_pass(sc_v, pm, lane, mv)

        g_wait(c3, g3, sem_g3, p)
        u = _weighted_pass(g3, sc_v, zinv, u)
        g_issue(c3, g3, sem_g3, pn)
        pltpu.async_copy(story_hbm.at[b2], idx2.at[p], sem_idx)

        for j in range(ND):
            uout_v[r, pl.ds(LANES * j, LANES)] = u[j]
        return carry

    lax.fori_loop(0, ROWS_PER, row_body, 0)

    g_wait(c1, g1, sem_g1, 0)
    g_wait(c2, g2, sem_g2, 0)
    g_wait(c3, g3, sem_g3, 0)
    idx_wait(base, 1)

    pltpu.sync_copy(uout_v, out_hbm.at[pl.ds(base, ROWS_PER)])


def kernel(story, C0, C1, C2, C3):
    del C0
    story_r = story.reshape(B, IDX_SPLIT, IDX_MINOR)
    return _encoder_sc(story_r, C1, C2, C3)


# speedup vs baseline: 1.0475x; 1.0475x over previous
import functools

import jax
import jax.numpy as jnp
from jax import lax
from jax.experimental import pallas as pl
from jax.experimental.pallas import tpu as pltpu
from jax.experimental.pallas import tpu_sc as plsc

B = 1024
L = 200
D = 128
LANES = 16
ND = D // LANES
NCHUNK = 13
NC = 2
NS = 16
NW = NC * NS
ROWS_PER = B // NW
IDX_SPLIT = 2
IDX_MINOR = L // IDX_SPLIT


def _mean_pass(g1):
    SUM_UNROLL = 4
    zeros8 = tuple(jnp.zeros((LANES,), jnp.float32) for _ in range(ND))

    @plsc.parallel_loop(0, L, SUM_UNROLL, unroll=2, carry=zeros8)
    def sum_body(l0, acc):
        for t in range(SUM_UNROLL):
            l = l0 + t
            acc = tuple(acc[j] + g1[l, pl.ds(LANES * j, LANES)]
                        for j in range(ND))
        return acc

    return tuple(x * jnp.float32(1.0 / L) for x in sum_body)


def _xlane_reduce(pm, lane, v, op):
    for s in (8, 4, 2, 1):
        plsc.store_scatter(pm, [lane], v)
        shuf = plsc.load_gather(pm, [lane ^ s])
        v = op(v, shuf)
    return v


def _score_pass(ga, u, sc_v, pm, lane):
    mneg = jnp.full((LANES,), -1e30, jnp.float32)

    @plsc.parallel_loop(0, NCHUNK, 1, carry=mneg)
    def score_chunk(c, mv):
        poff = c * (17 * LANES)
        for k in range(LANES):
            l = jnp.minimum(c * LANES + k, L - 1)
            p = ga[l, pl.ds(0, LANES)] * u[0]
            for j in range(1, ND):
                p = p + ga[l, pl.ds(LANES * j, LANES)] * u[j]
            plsc.store_scatter(pm, [lane + (17 * k) + poff], p)
        sv = plsc.load_gather(pm, [lane * 17 + poff])
        for cc in range(1, LANES):
            sv = sv + plsc.load_gather(pm, [lane * 17 + cc + poff])
        valid = (c * LANES + lane) < L
        sv = jnp.where(valid, sv, jnp.float32(-1e30))
        sc_v[pl.ds(c * LANES, LANES)] = sv
        return jnp.maximum(mv, sv)

    return score_chunk


def _exp_pass(sc_v, pm, lane, mv):
    m = _xlane_reduce(pm, lane, mv, jnp.maximum)

    @plsc.parallel_loop(0, NCHUNK, 1, unroll=2,
                        carry=jnp.zeros((LANES,), jnp.float32))
    def exp_chunk(c, zv):
        e = jnp.exp(sc_v[pl.ds(c * LANES, LANES)] - m)
        sc_v[pl.ds(c * LANES, LANES)] = e
        return zv + e

    return jnp.float32(1.0) / _xlane_reduce(pm, lane, exp_chunk, jnp.add)


def _weighted_pass(gc, sc_v, zinv, u):
    zeros8 = tuple(jnp.zeros((LANES,), jnp.float32) for _ in range(ND))

    @plsc.parallel_loop(0, NCHUNK, 1, carry=zeros8)
    def w_chunk(c, acc):
        e = sc_v[pl.ds(c * LANES, LANES)]
        for k in range(LANES):
            l = jnp.minimum(c * LANES + k, L - 1)
            w = e[k]
            acc = tuple(acc[j] + w * gc[l, pl.ds(LANES * j, LANES)]
                        for j in range(ND))
        return acc

    return tuple(u[j] + w_chunk[j] * zinv for j in range(ND))


@functools.partial(
    pl.kernel,
    mesh=plsc.VectorSubcoreMesh(core_axis_name="c", subcore_axis_name="s"),
    compiler_params=pltpu.CompilerParams(needs_layout_passes=False),
    out_type=jax.ShapeDtypeStruct((B, D), jnp.float32),
    scratch_types=[
        pltpu.VMEM((ROWS_PER, IDX_SPLIT, IDX_MINOR), jnp.int32),
        pltpu.VMEM((L, D), jnp.float32),
        pltpu.VMEM((L, D), jnp.float32),
        pltpu.VMEM((L, D), jnp.float32),
        pltpu.VMEM((NCHUNK * LANES,), jnp.float32),
        pltpu.VMEM((NCHUNK * LANES * 17,), jnp.float32),
        pltpu.VMEM((ROWS_PER, D), jnp.float32),
        pltpu.SemaphoreType.DMA,
        pltpu.SemaphoreType.DMA,
        pltpu.SemaphoreType.DMA,
    ],
)
def _encoder_sc(story_hbm, c1, c2, c3, out_hbm,
                idxa, g1, g2, g3, sc_v, pm, uout_v,
                sem_g1, sem_g2, sem_g3):
    wid = lax.axis_index("s") * NC + lax.axis_index("c")
    base = wid * ROWS_PER
    lane = lax.iota(jnp.int32, LANES)

    def g_issue(tab, g, sem, r):
        for h in range(IDX_SPLIT):
            pltpu.async_copy(
                tab.at[idxa.at[r, h]],
                g.at[pl.ds(h * IDX_MINOR, IDX_MINOR)], sem)

    def g_wait(tab, g, sem, r):
        for h in range(IDX_SPLIT):
            pltpu.make_async_copy(
                tab.at[idxa.at[r, h]],
                g.at[pl.ds(h * IDX_MINOR, IDX_MINOR)], sem).wait()

    pltpu.sync_copy(story_hbm.at[pl.ds(base, ROWS_PER)], idxa)
    g_issue(c1, g1, sem_g1, 0)
    g_issue(c2, g2, sem_g2, 0)
    g_issue(c3, g3, sem_g3, 0)

    def row_body(r, carry):
        rn = jnp.minimum(r + 1, ROWS_PER - 1)

        g_wait(c1, g1, sem_g1, r)
        u = _mean_pass(g1)
        mv = _score_pass(g1, u, sc_v, pm, lane)
        g_issue(c1, g1, sem_g1, rn)
        zinv = _exp_pass(sc_v, pm, lane, mv)

        g_wait(c2, g2, sem_g2, r)
        u = _weighted_pass(g2, sc_v, zinv, u)
        mv = _score_pass(g2, u, sc_v, pm, lane)
        g_issue(c2, g2, sem_g2, rn)
        zinv = _exp_pass(sc_v, pm, lane, mv)

        g_wait(c3, g3, sem_g3, r)
        u = _weighted_pass(g3, sc_v, zinv, u)
        g_issue(c3, g3, sem_g3, rn)

        for j in range(ND):
            uout_v[r, pl.ds(LANES * j, LANES)] = u[j]
        return carry

    lax.fori_loop(0, ROWS_PER, row_body, 0)

    g_wait(c1, g1, sem_g1, 0)
    g_wait(c2, g2, sem_g2, 0)
    g_wait(c3, g3, sem_g3, 0)

    pltpu.sync_copy(uout_v, out_hbm.at[pl.ds(base, ROWS_PER)])


def kernel(story, C0, C1, C2, C3):
    del C0
    story_r = story.reshape(B, IDX_SPLIT, IDX_MINOR)
    return _encoder_sc(story_r, C1, C2, C3)


# speedup vs baseline: 1.0556x; 1.0078x over previous
import functools

import jax
import jax.numpy as jnp
from jax import lax
from jax.experimental import pallas as pl
from jax.experimental.pallas import tpu as pltpu
from jax.experimental.pallas import tpu_sc as plsc

B = 1024
L = 200
D = 128
LANES = 16
ND = D // LANES
NCHUNK = 13
NC = 2
NS = 16
NW = NC * NS
ROWS_PER = B // NW
IDX_SPLIT = 2
IDX_MINOR = L // IDX_SPLIT


def _mean_pass(g1):
    SUM_UNROLL = 4
    zeros8 = tuple(jnp.zeros((LANES,), jnp.float32) for _ in range(ND))

    @plsc.parallel_loop(0, L, SUM_UNROLL, unroll=2, carry=zeros8)
    def sum_body(l0, acc):
        for t in range(SUM_UNROLL):
            l = l0 + t
            acc = tuple(acc[j] + g1[l, pl.ds(LANES * j, LANES)]
                        for j in range(ND))
        return acc

    return tuple(x * jnp.float32(1.0 / L) for x in sum_body)


def _xlane_reduce(pm, lane, v, op):
    for s in (8, 4, 2, 1):
        plsc.store_scatter(pm, [lane], v)
        shuf = plsc.load_gather(pm, [lane ^ s])
        v = op(v, shuf)
    return v


def _score_pass(ga, u, sc_v, pm, lane):
    mneg = jnp.full((LANES,), -1e30, jnp.float32)

    @plsc.parallel_loop(0, NCHUNK, 1, unroll=2, carry=mneg)
    def score_chunk(c, mv):
        poff = c * (17 * LANES)
        for k in range(LANES):
            l = jnp.minimum(c * LANES + k, L - 1)
            p = ga[l, pl.ds(0, LANES)] * u[0]
            for j in range(1, ND):
                p = p + ga[l, pl.ds(LANES * j, LANES)] * u[j]
            plsc.store_scatter(pm, [lane + (17 * k) + poff], p)
        sv = plsc.load_gather(pm, [lane * 17 + poff])
        for cc in range(1, LANES):
            sv = sv + plsc.load_gather(pm, [lane * 17 + cc + poff])
        valid = (c * LANES + lane) < L
        sv = jnp.where(valid, sv, jnp.float32(-1e30))
        sc_v[pl.ds(c * LANES, LANES)] = sv
        return jnp.maximum(mv, sv)

    return score_chunk


def _exp_pass(sc_v, pm, lane, mv):
    m = _xlane_reduce(pm, lane, mv, jnp.maximum)

    @plsc.parallel_loop(0, NCHUNK, 1, unroll=2,
                        carry=jnp.zeros((LANES,), jnp.float32))
    def exp_chunk(c, zv):
        e = jnp.exp(sc_v[pl.ds(c * LANES, LANES)] - m)
        sc_v[pl.ds(c * LANES, LANES)] = e
        return zv + e

    return jnp.float32(1.0) / _xlane_reduce(pm, lane, exp_chunk, jnp.add)


def _weighted_pass(gc, sc_v, zinv, u):
    zeros8 = tuple(jnp.zeros((LANES,), jnp.float32) for _ in range(ND))

    @plsc.parallel_loop(0, NCHUNK, 1, unroll=2, carry=zeros8)
    def w_chunk(c, acc):
        e = sc_v[pl.ds(c * LANES, LANES)]
        for k in range(LANES):
            l = jnp.minimum(c * LANES + k, L - 1)
            w = e[k]
            acc = tuple(acc[j] + w * gc[l, pl.ds(LANES * j, LANES)]
                        for j in range(ND))
        return acc

    return tuple(u[j] + w_chunk[j] * zinv for j in range(ND))


@functools.partial(
    pl.kernel,
    mesh=plsc.VectorSubcoreMesh(core_axis_name="c", subcore_axis_name="s"),
    compiler_params=pltpu.CompilerParams(needs_layout_passes=False),
    out_type=jax.ShapeDtypeStruct((B, D), jnp.float32),
    scratch_types=[
        pltpu.VMEM((ROWS_PER, IDX_SPLIT, IDX_MINOR), jnp.int32),
        pltpu.VMEM((L, D), jnp.float32),
        pltpu.VMEM((L, D), jnp.float32),
        pltpu.VMEM((L, D), jnp.float32),
        pltpu.VMEM((NCHUNK * LANES,), jnp.float32),
        pltpu.VMEM((NCHUNK * LANES * 17,), jnp.float32),
        pltpu.VMEM((ROWS_PER, D), jnp.float32),
        pltpu.SemaphoreType.DMA,
        pltpu.SemaphoreType.DMA,
        pltpu.SemaphoreType.DMA,
    ],
)
def _encoder_sc(story_hbm, c1, c2, c3, out_hbm,
                idxa, g1, g2, g3, sc_v, pm, uout_v,
                sem_g1, sem_g2, sem_g3):
    wid = lax.axis_index("s") * NC + lax.axis_index("c")
    base = wid * ROWS_PER
    lane = lax.iota(jnp.int32, LANES)

    def g_issue(tab, g, sem, r):
        for h in range(IDX_SPLIT):
            pltpu.async_copy(
                tab.at[idxa.at[r, h]],
                g.at[pl.ds(h * IDX_MINOR, IDX_MINOR)], sem)

    def g_wait(tab, g, sem, r):
        for h in range(IDX_SPLIT):
            pltpu.make_async_copy(
                tab.at[idxa.at[r, h]],
                g.at[pl.ds(h * IDX_MINOR, IDX_MINOR)], sem).wait()

    pltpu.sync_copy(story_hbm.at[pl.ds(base, ROWS_PER)], idxa)
    g_issue(c1, g1, sem_g1, 0)
    g_issue(c2, g2, sem_g2, 0)
    g_issue(c3, g3, sem_g3, 0)

    def row_body(r, carry):
        rn = jnp.minimum(r + 1, ROWS_PER - 1)

        g_wait(c1, g1, sem_g1, r)
        u = _mean_pass(g1)
        mv = _score_pass(g1, u, sc_v, pm, lane)
        g_issue(c1, g1, sem_g1, rn)
        zinv = _exp_pass(sc_v, pm, lane, mv)

        g_wait(c2, g2, sem_g2, r)
        u = _weighted_pass(g2, sc_v, zinv, u)
        mv = _score_pass(g2, u, sc_v, pm, lane)
        g_issue(c2, g2, sem_g2, rn)
        zinv = _exp_pass(sc_v, pm, lane, mv)

        g_wait(c3, g3, sem_g3, r)
        u = _weighted_pass(g3, sc_v, zinv, u)
        g_issue(c3, g3, sem_g3, rn)

        for j in range(ND):
            uout_v[r, pl.ds(LANES * j, LANES)] = u[j]
        return carry

    lax.fori_loop(0, ROWS_PER, row_body, 0)

    g_wait(c1, g1, sem_g1, 0)
    g_wait(c2, g2, sem_g2, 0)
    g_wait(c3, g3, sem_g3, 0)

    pltpu.sync_copy(uout_v, out_hbm.at[pl.ds(base, ROWS_PER)])


def kernel(story, C0, C1, C2, C3):
    del C0
    story_r = story.reshape(B, IDX_SPLIT, IDX_MINOR)
    return _encoder_sc(story_r, C1, C2, C3)
